# baseline (speedup 1.0000x reference)
import functools
import math

import jax
import jax.numpy as jnp
from jax.experimental import pallas as pl
from jax.experimental.pallas import tpu as pltpu

_SQRT_2_OVER_PI = math.sqrt(2.0 / math.pi)

_CHUNK = 512
_NBUF = 6
_TAPER = 128
_NT = _CHUNK // _TAPER


def _gelu(x):
    inner = _SQRT_2_OVER_PI * (x + 0.044715 * (x * x * x))
    return 0.5 * x * (1.0 + jnp.tanh(inner))


def _body(x_hbm, o_hbm, xbuf, ybuf, in_sem, out_sem, tin_sem, tout_sem, *,
          n_chunks):
    def copy_in(i, slot):
        return pltpu.make_async_copy(
            x_hbm.at[pl.ds(i * _CHUNK, _CHUNK), :], xbuf.at[slot], in_sem.at[slot])

    def copy_out(i, slot):
        return pltpu.make_async_copy(
            ybuf.at[slot], o_hbm.at[pl.ds(i * _CHUNK, _CHUNK), :], out_sem.at[slot])

    def taper_in(j):
        return pltpu.make_async_copy(
            x_hbm.at[pl.ds(j * _TAPER, _TAPER), :],
            xbuf.at[0, pl.ds(j * _TAPER, _TAPER), :],
            tin_sem.at[j])

    last = n_chunks - 1
    lslot = last % _NBUF

    def taper_out(j):
        return pltpu.make_async_copy(
            ybuf.at[lslot, pl.ds(j * _TAPER, _TAPER), :],
            o_hbm.at[pl.ds(last * _CHUNK + j * _TAPER, _TAPER), :],
            tout_sem.at[j])

    for j in range(_NT):
        taper_in(j).start()
    for c in range(1, _NBUF):
        copy_in(c, c).start()
    for j in range(_NT):
        taper_in(j).wait()
        ybuf[0, j * _TAPER:(j + 1) * _TAPER, :] = _gelu(
            xbuf[0, j * _TAPER:(j + 1) * _TAPER, :])
    copy_out(0, 0).start()
    copy_in(_NBUF, 0).start()

    def step(c, carry):
        slot = jax.lax.rem(c, _NBUF)
        copy_in(c, slot).wait()

        @pl.when(c >= _NBUF)
        def _():
            copy_out(c - _NBUF, slot).wait()

        ybuf[slot] = _gelu(xbuf[slot])
        copy_out(c, slot).start()

        @pl.when(c + _NBUF < n_chunks)
        def _():
            copy_in(c + _NBUF, slot).start()

        return carry

    jax.lax.fori_loop(1, last, step, 0)

    copy_in(last, lslot).wait()
    copy_out(last - _NBUF, lslot).wait()
    for j in range(_NT):
        ybuf[lslot, j * _TAPER:(j + 1) * _TAPER, :] = _gelu(
            xbuf[lslot, j * _TAPER:(j + 1) * _TAPER, :])
        taper_out(j).start()
    for c in range(n_chunks - _NBUF, last):
        copy_out(c, c % _NBUF).wait()
    for j in range(_NT):
        taper_out(j).wait()


def kernel(x, log_k):
    B, T, D = x.shape
    rows = B * T
    x2 = x.reshape(rows, D)
    n_chunks = rows // _CHUNK
    y2 = pl.pallas_call(
        functools.partial(_body, n_chunks=n_chunks),
        in_specs=[pl.BlockSpec(memory_space=pl.ANY)],
        out_specs=pl.BlockSpec(memory_space=pl.ANY),
        out_shape=jax.ShapeDtypeStruct((rows, D), x.dtype),
        scratch_shapes=[
            pltpu.VMEM((_NBUF, _CHUNK, D), x.dtype),
            pltpu.VMEM((_NBUF, _CHUNK, D), x.dtype),
            pltpu.SemaphoreType.DMA((_NBUF,)),
            pltpu.SemaphoreType.DMA((_NBUF,)),
            pltpu.SemaphoreType.DMA((_NT,)),
            pltpu.SemaphoreType.DMA((_NT,)),
        ],
    )(x2)
    return y2.reshape(B, T, D)
